# baseline (speedup 1.0000x reference)
import functools

import jax
import jax.numpy as jnp
from jax.experimental import pallas as pl
from jax.experimental.pallas import tpu as pltpu

_A = 9


def _rpn_body(offsets, P0, NP, flat_ref, w1_ref, b1_ref, w2_ref, b2_ref,
              wcat_ref, bcat_ref, out_ref, s2_ref):
    C = w1_ref.shape[1]
    Wp = 66
    total = s2_ref.shape[0]

    acc = jnp.zeros((NP, C), dtype=jnp.float32)
    for k, o in enumerate(offsets):
        acc += jnp.dot(flat_ref[0, pl.ds(P0 + o, NP), :], w1_ref[k],
                       preferred_element_type=jnp.float32)
    h1 = jnp.maximum(acc + b1_ref[0], 0.0)

    col = jax.lax.broadcasted_iota(jnp.int32, (NP, 1), 0) % Wp
    valid = (col >= 1) & (col <= Wp - 2)
    h1 = jnp.where(valid, h1, 0.0)

    s2_ref[pl.ds(0, P0), :] = jnp.zeros((P0, C), dtype=jnp.float32)
    s2_ref[pl.ds(P0 + NP, total - P0 - NP), :] = jnp.zeros(
        (total - P0 - NP, C), dtype=jnp.float32)
    s2_ref[pl.ds(P0, NP), :] = h1

    acc2 = jnp.zeros((NP, C), dtype=jnp.float32)
    for k, o in enumerate(offsets):
        acc2 += jnp.dot(s2_ref[pl.ds(P0 + o, NP), :], w2_ref[k],
                        preferred_element_type=jnp.float32)
    h2 = jnp.maximum(acc2 + b2_ref[0], 0.0)

    z = jnp.dot(h2, wcat_ref[...], preferred_element_type=jnp.float32)
    z = z + bcat_ref[0]
    ch = jax.lax.broadcasted_iota(jnp.int32, (1, 6 * _A), 1) % 6
    is_cls = ch < 2
    out_ref[0] = jnp.where(is_cls, 1.0 / (1.0 + jnp.exp(-z)), z)


def kernel(input, W1, b1, W2, b2, Wreg, breg, Wcls, bcls):
    B, H, W, C = input.shape
    A = _A
    Hp, Wp = H + 4, W + 2
    NP = H * Wp
    P0 = 2 * Wp

    xp = jnp.pad(input, ((0, 0), (2, 2), (1, 1), (0, 0)))
    flat = xp.reshape(B, Hp * Wp, C)

    W1f = W1.reshape(9, C, C)
    W2f = W2.reshape(9, C, C)

    Wc = Wcls.reshape(C, A, 2)
    d0 = (Wc[:, :, 0] - Wc[:, :, 1])[:, :, None]
    Wcat = jnp.concatenate([d0, -d0, Wreg.reshape(C, A, 4)],
                           axis=2).reshape(1 * C, 6 * A)
    bc = bcls.reshape(A, 2)
    bd0 = (bc[:, 0] - bc[:, 1])[:, None]
    bcat = jnp.concatenate([bd0, -bd0, breg.reshape(A, 4)],
                           axis=1).reshape(1, 6 * A)

    offsets = [(a - 1) * Wp + (b - 1) for a in range(3) for b in range(3)]

    out = pl.pallas_call(
        functools.partial(_rpn_body, offsets, P0, NP),
        grid=(B,),
        in_specs=[
            pl.BlockSpec((1, Hp * Wp, C), lambda b: (b, 0, 0)),
            pl.BlockSpec((9, C, C), lambda b: (0, 0, 0)),
            pl.BlockSpec((1, C), lambda b: (0, 0)),
            pl.BlockSpec((9, C, C), lambda b: (0, 0, 0)),
            pl.BlockSpec((1, C), lambda b: (0, 0)),
            pl.BlockSpec((C, 6 * A), lambda b: (0, 0)),
            pl.BlockSpec((1, 6 * A), lambda b: (0, 0)),
        ],
        out_specs=pl.BlockSpec((1, NP, 6 * A), lambda b: (b, 0, 0)),
        out_shape=jax.ShapeDtypeStruct((B, NP, 6 * A), jnp.float32),
        scratch_shapes=[pltpu.VMEM((Hp * Wp, C), jnp.float32)],
    )(flat, W1f, b1.reshape(1, C), W2f, b2.reshape(1, C), Wcat, bcat)

    return out.reshape(B, H, Wp, 6 * A)[:, :, 1:W + 1, :].reshape(
        B, H, W, A, 6)


# speedup vs baseline: 1.1240x; 1.1240x over previous
import functools

import jax
import jax.numpy as jnp
from jax.experimental import pallas as pl
from jax.experimental.pallas import tpu as pltpu

_A = 9


def _rpn_body(offsets, P0, NP, flat_ref, w1_ref, b1_ref, w2_ref, b2_ref,
              wcat_ref, bcat_ref, out_ref, s2_ref):
    C = w1_ref.shape[1]
    Wp = 66
    total = s2_ref.shape[0]

    acc = jnp.zeros((NP, C), dtype=jnp.float32)
    for k, o in enumerate(offsets):
        acc += jnp.dot(flat_ref[0, pl.ds(P0 + o, NP), :], w1_ref[k],
                       preferred_element_type=jnp.float32)
    h1 = jnp.maximum(acc + b1_ref[0], 0.0)

    col = jax.lax.broadcasted_iota(jnp.int32, (NP, 1), 0) % Wp
    valid = (col >= 1) & (col <= Wp - 2)
    h1 = jnp.where(valid, h1, 0.0).astype(jnp.bfloat16)

    s2_ref[pl.ds(0, P0), :] = jnp.zeros((P0, C), dtype=jnp.bfloat16)
    s2_ref[pl.ds(P0 + NP, total - P0 - NP), :] = jnp.zeros(
        (total - P0 - NP, C), dtype=jnp.bfloat16)
    s2_ref[pl.ds(P0, NP), :] = h1

    acc2 = jnp.zeros((NP, C), dtype=jnp.float32)
    for k, o in enumerate(offsets):
        acc2 += jnp.dot(s2_ref[pl.ds(P0 + o, NP), :], w2_ref[k],
                        preferred_element_type=jnp.float32)
    h2 = jnp.maximum(acc2 + b2_ref[0], 0.0).astype(jnp.bfloat16)

    z = jnp.dot(h2, wcat_ref[...], preferred_element_type=jnp.float32)
    z = z + bcat_ref[0]
    ch = jax.lax.broadcasted_iota(jnp.int32, (1, 6 * _A), 1) % 6
    is_cls = ch < 2
    out_ref[0] = jnp.where(is_cls, 1.0 / (1.0 + jnp.exp(-z)), z)


def kernel(input, W1, b1, W2, b2, Wreg, breg, Wcls, bcls):
    B, H, W, C = input.shape
    A = _A
    Hp, Wp = H + 4, W + 2
    NP = H * Wp
    P0 = 2 * Wp

    xp = jnp.pad(input, ((0, 0), (2, 2), (1, 1), (0, 0)))
    flat = xp.reshape(B, Hp * Wp, C).astype(jnp.bfloat16)

    W1f = W1.reshape(9, C, C).astype(jnp.bfloat16)
    W2f = W2.reshape(9, C, C).astype(jnp.bfloat16)

    Wc = Wcls.reshape(C, A, 2)
    d0 = (Wc[:, :, 0] - Wc[:, :, 1])[:, :, None]
    Wcat = jnp.concatenate([d0, -d0, Wreg.reshape(C, A, 4)],
                           axis=2).reshape(1 * C, 6 * A).astype(jnp.bfloat16)
    bc = bcls.reshape(A, 2)
    bd0 = (bc[:, 0] - bc[:, 1])[:, None]
    bcat = jnp.concatenate([bd0, -bd0, breg.reshape(A, 4)],
                           axis=1).reshape(1, 6 * A)

    offsets = [(a - 1) * Wp + (b - 1) for a in range(3) for b in range(3)]

    out = pl.pallas_call(
        functools.partial(_rpn_body, offsets, P0, NP),
        grid=(B,),
        in_specs=[
            pl.BlockSpec((1, Hp * Wp, C), lambda b: (b, 0, 0)),
            pl.BlockSpec((9, C, C), lambda b: (0, 0, 0)),
            pl.BlockSpec((1, C), lambda b: (0, 0)),
            pl.BlockSpec((9, C, C), lambda b: (0, 0, 0)),
            pl.BlockSpec((1, C), lambda b: (0, 0)),
            pl.BlockSpec((C, 6 * A), lambda b: (0, 0)),
            pl.BlockSpec((1, 6 * A), lambda b: (0, 0)),
        ],
        out_specs=pl.BlockSpec((1, NP, 6 * A), lambda b: (b, 0, 0)),
        out_shape=jax.ShapeDtypeStruct((B, NP, 6 * A), jnp.float32),
        scratch_shapes=[pltpu.VMEM((Hp * Wp, C), jnp.bfloat16)],
    )(flat, W1f, b1.reshape(1, C), W2f, b2.reshape(1, C), Wcat, bcat)

    return out.reshape(B, H, Wp, 6 * A)[:, :, 1:W + 1, :].reshape(
        B, H, W, A, 6)


# speedup vs baseline: 1.4979x; 1.3327x over previous
import functools

import jax
import jax.numpy as jnp
from jax.experimental import pallas as pl
from jax.experimental.pallas import tpu as pltpu

_A = 9
_G = 16


def _rpn_body(H, W, flat_ref, w1_ref, b1_ref, w2_ref, b2_ref,
              wcat_ref, bcat_ref, out_ref, ls_ref, rs_ref, s2_ref):
    C = w1_ref.shape[1]
    XQ = (H + 2) * W
    NP = H * W
    wpos = jax.lax.broadcasted_iota(jnp.int32, (XQ, 1), 0) % W
    zero = jnp.zeros((), dtype=jnp.bfloat16)

    left = flat_ref[0, pl.ds(_G - 1, XQ), :]
    ls_ref[pl.ds(_G, XQ), :] = jnp.where(wpos == 0, zero, left)
    right = flat_ref[0, pl.ds(_G + 1, XQ), :]
    rs_ref[pl.ds(_G, XQ), :] = jnp.where(wpos == W - 1, zero, right)

    acc = jnp.zeros((NP, C), dtype=jnp.float32)
    for dh in range(3):
        s = _G + dh * W
        acc += jnp.dot(ls_ref[pl.ds(s, NP), :], w1_ref[dh * 3 + 0],
                       preferred_element_type=jnp.float32)
        acc += jnp.dot(flat_ref[0, pl.ds(s, NP), :], w1_ref[dh * 3 + 1],
                       preferred_element_type=jnp.float32)
        acc += jnp.dot(rs_ref[pl.ds(s, NP), :], w1_ref[dh * 3 + 2],
                       preferred_element_type=jnp.float32)
    h1 = jnp.maximum(acc + b1_ref[0], 0.0).astype(jnp.bfloat16)

    s2_ref[pl.ds(0, _G + W), :] = jnp.zeros((_G + W, C), dtype=jnp.bfloat16)
    s2_ref[pl.ds(_G + W + NP, _G + W), :] = jnp.zeros((_G + W, C),
                                                      dtype=jnp.bfloat16)
    s2_ref[pl.ds(_G + W, NP), :] = h1

    left = s2_ref[pl.ds(_G - 1, XQ), :]
    ls_ref[pl.ds(_G, XQ), :] = jnp.where(wpos == 0, zero, left)
    right = s2_ref[pl.ds(_G + 1, XQ), :]
    rs_ref[pl.ds(_G, XQ), :] = jnp.where(wpos == W - 1, zero, right)

    acc2 = jnp.zeros((NP, C), dtype=jnp.float32)
    for dh in range(3):
        s = _G + dh * W
        acc2 += jnp.dot(ls_ref[pl.ds(s, NP), :], w2_ref[dh * 3 + 0],
                        preferred_element_type=jnp.float32)
        acc2 += jnp.dot(s2_ref[pl.ds(s, NP), :], w2_ref[dh * 3 + 1],
                        preferred_element_type=jnp.float32)
        acc2 += jnp.dot(rs_ref[pl.ds(s, NP), :], w2_ref[dh * 3 + 2],
                        preferred_element_type=jnp.float32)
    h2 = jnp.maximum(acc2 + b2_ref[0], 0.0).astype(jnp.bfloat16)

    z = jnp.dot(h2, wcat_ref[...], preferred_element_type=jnp.float32)
    z = z + bcat_ref[0]
    ch = jax.lax.broadcasted_iota(jnp.int32, (1, 6 * _A), 1) % 6
    out_ref[0] = jnp.where(ch < 2, 1.0 / (1.0 + jnp.exp(-z)), z)


def kernel(input, W1, b1, W2, b2, Wreg, breg, Wcls, bcls):
    B, H, W, C = input.shape
    A = _A
    NP = H * W
    XQ = (H + 2) * W
    TOT = XQ + 2 * _G

    xp = jnp.pad(input, ((0, 0), (1, 1), (0, 0), (0, 0)))
    flat = jnp.pad(xp.reshape(B, XQ, C), ((0, 0), (_G, _G), (0, 0)))
    flat = flat.astype(jnp.bfloat16)

    W1f = W1.reshape(9, C, C).astype(jnp.bfloat16)
    W2f = W2.reshape(9, C, C).astype(jnp.bfloat16)

    Wc = Wcls.reshape(C, A, 2)
    d0 = (Wc[:, :, 0] - Wc[:, :, 1])[:, :, None]
    Wcat = jnp.concatenate([d0, -d0, Wreg.reshape(C, A, 4)],
                           axis=2).reshape(C, 6 * A).astype(jnp.bfloat16)
    bc = bcls.reshape(A, 2)
    bd0 = (bc[:, 0] - bc[:, 1])[:, None]
    bcat = jnp.concatenate([bd0, -bd0, breg.reshape(A, 4)],
                           axis=1).reshape(1, 6 * A)

    out = pl.pallas_call(
        functools.partial(_rpn_body, H, W),
        grid=(B,),
        in_specs=[
            pl.BlockSpec((1, TOT, C), lambda b: (b, 0, 0)),
            pl.BlockSpec((9, C, C), lambda b: (0, 0, 0)),
            pl.BlockSpec((1, C), lambda b: (0, 0)),
            pl.BlockSpec((9, C, C), lambda b: (0, 0, 0)),
            pl.BlockSpec((1, C), lambda b: (0, 0)),
            pl.BlockSpec((C, 6 * A), lambda b: (0, 0)),
            pl.BlockSpec((1, 6 * A), lambda b: (0, 0)),
        ],
        out_specs=pl.BlockSpec((1, NP, 6 * A), lambda b: (b, 0, 0)),
        out_shape=jax.ShapeDtypeStruct((B, NP, 6 * A), jnp.float32),
        scratch_shapes=[
            pltpu.VMEM((TOT, C), jnp.bfloat16),
            pltpu.VMEM((TOT, C), jnp.bfloat16),
            pltpu.VMEM((TOT, C), jnp.bfloat16),
        ],
    )(flat, W1f, b1.reshape(1, C), W2f, b2.reshape(1, C), Wcat, bcat)

    return out.reshape(B, H, W, A, 6)
